# Initial kernel scaffold; baseline (speedup 1.0000x reference)
import jax
import jax.numpy as jnp
from jax.experimental import pallas as pl


def kernel(x, edge_index, edge_weight, W1, b1, W2, b2, W_out, b_out):
    raise NotImplementedError("write your pallas kernel here")


# speedup vs baseline: 3.1142x; 3.1142x over previous
import functools

import jax
import jax.numpy as jnp
from jax import lax
from jax.experimental import pallas as pl
from jax.experimental.pallas import tpu as pltpu
from jax.experimental.pallas import tpu_sc as plsc

_MIN_NORM = 1e-15
_EPS = 4e-3


def _artanh(x):
    x = jnp.clip(x, -1.0 + 1e-7, 1.0 - 1e-7)
    return 0.5 * jnp.log((1.0 + x) / (1.0 - x))


def _norm(x):
    return jnp.maximum(jnp.sqrt(jnp.sum(x * x, axis=-1, keepdims=True)), _MIN_NORM)


def _proj(x):
    n = _norm(x)
    maxnorm = 1.0 - _EPS
    return jnp.where(n > maxnorm, x / n * maxnorm, x)


def _expmap0(u):
    n = _norm(u)
    return jnp.tanh(n) * u / n


def _logmap0(p):
    n = _norm(p)
    return _artanh(n) * p / n


def _mobius_add(x, y):
    x2 = jnp.sum(x * x, axis=-1, keepdims=True)
    y2 = jnp.sum(y * y, axis=-1, keepdims=True)
    xy = jnp.sum(x * y, axis=-1, keepdims=True)
    num = (1.0 + 2.0 * xy + y2) * x + (1.0 - x2) * y
    den = 1.0 + 2.0 * xy + x2 * y2
    return num / jnp.maximum(den, _MIN_NORM)


def _mobius_matvec(wt, x):
    xn = _norm(x)
    mx = jnp.dot(x, wt, preferred_element_type=jnp.float32)
    mxn = _norm(mx)
    res = jnp.tanh(mxn / xn * _artanh(xn)) * mx / mxn
    cond = jnp.all(mx == 0.0, axis=-1, keepdims=True)
    return jnp.where(cond, 0.0, res)


def _hyplinear_logmap(h, wt, b_row):
    mv = _proj(_mobius_matvec(wt, h))
    hb = _proj(_expmap0(b_row))
    res = _proj(_mobius_add(mv, hb))
    return _logmap0(res)


def _pre1_body(x_ref, w_ref, b_ref, o_ref):
    o_ref[...] = _hyplinear_logmap(x_ref[...], w_ref[...], b_ref[...])


def _mid_body(a_ref, w_ref, b_ref, o_ref):
    h2 = _proj(_expmap0(a_ref[0] + a_ref[1]))
    t = jnp.maximum(_logmap0(h2), 0.0)
    h = _proj(_expmap0(t))
    o_ref[...] = _hyplinear_logmap(h, w_ref[...], b_ref[...])


def _post_body(a_ref, w_ref, b_ref, o_ref):
    h2 = _proj(_expmap0(a_ref[...]))
    t = _logmap0(h2)
    h = _proj(_expmap0(t))
    ht = _logmap0(h)
    o_ref[...] = (
        jnp.dot(ht, w_ref[...], preferred_element_type=jnp.float32) + b_ref[...]
    )


def _tc_stage(body, x, wt, b_row, d_out, blk=1000):
    if x.ndim == 3:
        n = x.shape[1]
        x_spec = pl.BlockSpec((2, blk, x.shape[2]), lambda i: (0, i, 0))
    else:
        n = x.shape[0]
        x_spec = pl.BlockSpec((blk, x.shape[1]), lambda i: (i, 0))
    return pl.pallas_call(
        body,
        grid=(n // blk,),
        in_specs=[
            x_spec,
            pl.BlockSpec(wt.shape, lambda i: (0, 0)),
            pl.BlockSpec(b_row.shape, lambda i: (0, 0)),
        ],
        out_specs=pl.BlockSpec((blk, d_out), lambda i: (i, 0)),
        out_shape=jax.ShapeDtypeStruct((n, d_out), jnp.float32),
    )(x, wt, b_row)


_CHUNK = 80
_NSUB = 16


@functools.cache
def _make_spmm(n_nodes, n_edges, dc, feat_split):
    chunk = _CHUNK
    nsub = _NSUB
    nvec = dc // 16
    n_chunks = n_edges // chunk
    assert n_edges % chunk == 0 and dc % 128 == 0
    if feat_split:
        per_sub = n_edges // nsub
        assert per_sub % chunk == 0
    else:
        base_iters = n_chunks // (2 * nsub)
        extra = n_chunks - base_iters * 2 * nsub
        assert extra in (0, nsub)
    rpt = ((n_nodes // nsub) // 8) * 8
    last_extra = n_nodes - rpt * nsub
    assert 0 <= last_extra < rpt and last_extra % 8 == 0
    assert rpt % chunk != 0 or True
    mesh = plsc.VectorSubcoreMesh(
        core_axis_name="c", subcore_axis_name="s", num_cores=2, num_subcores=nsub
    )

    @functools.partial(
        pl.kernel,
        out_type=jax.ShapeDtypeStruct((2, n_nodes, dc), jnp.float32),
        mesh=mesh,
        scratch_types=[
            pltpu.VMEM((chunk,), jnp.int32),
            pltpu.VMEM((chunk,), jnp.int32),
            pltpu.VMEM((chunk,), jnp.float32),
            pltpu.VMEM((chunk, dc), jnp.float32),
            pltpu.VMEM_SHARED((n_nodes, dc), jnp.float32),
            pltpu.SemaphoreType.DMA,
        ],
    )
    def spmm(xt0_hbm, xt1_hbm, src_hbm, dst_hbm, w_hbm, out_hbm,
             src_v, dst_v, w_v, rows_v, acc, sem):
        c = lax.axis_index("c")
        s = lax.axis_index("s")
        start = s * rpt

        def zrow(i, carry):
            for j in range(nvec):
                rows_v[i, pl.ds(j * 16, 16)] = jnp.zeros((16,), jnp.float32)
            return carry

        lax.fori_loop(0, chunk, zrow, 0)
        nfull = rpt // chunk
        for t in range(nfull):
            pltpu.sync_copy(rows_v, acc.at[pl.ds(start + t * chunk, chunk)])
        tail = rpt - nfull * chunk

        if tail:
            @pl.when(s < nsub - 1)
            def _():
                pltpu.sync_copy(rows_v.at[pl.ds(0, tail)],
                                acc.at[pl.ds(start + nfull * chunk, tail)])
        ltail = rpt + last_extra - nfull * chunk

        @pl.when(s == nsub - 1)
        def _():
            done = nfull * chunk
            for t in range(ltail // chunk):
                pltpu.sync_copy(rows_v,
                                acc.at[pl.ds(start + done + t * chunk, chunk)])
            rem = ltail - (ltail // chunk) * chunk
            if rem:
                pltpu.sync_copy(rows_v.at[pl.ds(0, rem)],
                                acc.at[pl.ds(start + done + (ltail // chunk) * chunk,
                                             rem)])
        plsc.subcore_barrier()

        def step(off):
            pltpu.sync_copy(src_hbm.at[pl.ds(off, chunk)], src_v)
            pltpu.sync_copy(dst_hbm.at[pl.ds(off, chunk)], dst_v)
            pltpu.sync_copy(w_hbm.at[pl.ds(off, chunk)], w_v)

            if feat_split:
                @pl.when(c == 0)
                def _():
                    pltpu.async_copy(xt0_hbm.at[src_v], rows_v, sem).wait()

                @pl.when(c == 1)
                def _():
                    pltpu.async_copy(xt1_hbm.at[src_v], rows_v, sem).wait()
            else:
                pltpu.async_copy(xt0_hbm.at[src_v], rows_v, sem).wait()

            def scale(g, carry2):
                wg = w_v[pl.ds(g * 16, 16)]
                base = g * 16
                for r16 in range(16):
                    wsp = jnp.broadcast_to(wg[r16], (16,))
                    for j in range(nvec):
                        rows_v[base + r16, pl.ds(j * 16, 16)] = (
                            rows_v[base + r16, pl.ds(j * 16, 16)] * wsp
                        )
                return carry2

            lax.fori_loop(0, chunk // 16, scale, 0)
            pltpu.sync_copy(rows_v, acc.at[dst_v], add=True)

        if feat_split:
            def body(i, carry):
                step((s * (n_edges // nsub) // chunk + i) * chunk)
                return carry

            lax.fori_loop(0, n_edges // nsub // chunk, body, 0)
        else:
            w_id = c * nsub + s

            def body(i, carry):
                step((w_id + i * 2 * nsub) * chunk)
                return carry

            n_i = base_iters + (1 if extra else 0) * jnp.where(c == 0, 1, 0)
            lax.fori_loop(0, n_i, body, 0)
        plsc.subcore_barrier()

        for cc in range(2):
            @pl.when(jnp.logical_and(c == cc, s < nsub - 1))
            def _(cc=cc):
                pltpu.sync_copy(acc.at[pl.ds(start, rpt)],
                                out_hbm.at[cc, pl.ds(start, rpt)])

            @pl.when(jnp.logical_and(c == cc, s == nsub - 1))
            def _(cc=cc):
                pltpu.sync_copy(acc.at[pl.ds(start, rpt + last_extra)],
                                out_hbm.at[cc, pl.ds(start, rpt + last_extra)])

    return spmm


def _spmm_feat_split(xt, src, dst, w):
    n, d = xt.shape
    dc = d // 2
    f = _make_spmm(n, src.shape[0], dc, True)
    out = f(xt[:, :dc], xt[:, dc:], src, dst, w)
    return out.transpose(1, 0, 2).reshape(n, d)


def _spmm_edge_split(xt, src, dst, w):
    n, d = xt.shape
    f = _make_spmm(n, src.shape[0], d, False)
    return f(xt, xt, src, dst, w)


def kernel(x, edge_index, edge_weight, W1, b1, W2, b2, W_out, b_out):
    d_hid = W1.shape[0]
    d_feat = W2.shape[0]
    n_out = W_out.shape[0]
    src = edge_index[0].astype(jnp.int32)
    dst = edge_index[1].astype(jnp.int32)

    xt1 = _tc_stage(_pre1_body, x, W1.T, b1.reshape(1, -1), d_hid)
    agg1 = _spmm_edge_split(xt1, src, dst, edge_weight)
    xt2 = _tc_stage(_mid_body, agg1, W2.T, b2.reshape(1, -1), d_feat)
    agg2 = _spmm_feat_split(xt2, src, dst, edge_weight)
    return _tc_stage(_post_body, agg2, W_out.T, b_out.reshape(1, -1), n_out)


# speedup vs baseline: 4.9431x; 1.5873x over previous
import functools

import jax
import jax.numpy as jnp
from jax import lax
from jax.experimental import pallas as pl
from jax.experimental.pallas import tpu as pltpu
from jax.experimental.pallas import tpu_sc as plsc

_MIN_NORM = 1e-15
_EPS = 4e-3


def _artanh(x):
    x = jnp.clip(x, -1.0 + 1e-7, 1.0 - 1e-7)
    return 0.5 * jnp.log((1.0 + x) / (1.0 - x))


def _norm(x):
    return jnp.maximum(jnp.sqrt(jnp.sum(x * x, axis=-1, keepdims=True)), _MIN_NORM)


def _proj(x):
    n = _norm(x)
    maxnorm = 1.0 - _EPS
    return jnp.where(n > maxnorm, x / n * maxnorm, x)


def _expmap0(u):
    n = _norm(u)
    return jnp.tanh(n) * u / n


def _logmap0(p):
    n = _norm(p)
    return _artanh(n) * p / n


def _mobius_add(x, y):
    x2 = jnp.sum(x * x, axis=-1, keepdims=True)
    y2 = jnp.sum(y * y, axis=-1, keepdims=True)
    xy = jnp.sum(x * y, axis=-1, keepdims=True)
    num = (1.0 + 2.0 * xy + y2) * x + (1.0 - x2) * y
    den = 1.0 + 2.0 * xy + x2 * y2
    return num / jnp.maximum(den, _MIN_NORM)


def _mobius_matvec(wt, x):
    xn = _norm(x)
    mx = jnp.dot(x, wt, preferred_element_type=jnp.float32)
    mxn = _norm(mx)
    res = jnp.tanh(mxn / xn * _artanh(xn)) * mx / mxn
    cond = jnp.all(mx == 0.0, axis=-1, keepdims=True)
    return jnp.where(cond, 0.0, res)


def _hyplinear_logmap(h, wt, b_row):
    mv = _proj(_mobius_matvec(wt, h))
    hb = _proj(_expmap0(b_row))
    res = _proj(_mobius_add(mv, hb))
    return _logmap0(res)


def _pre1_body(x_ref, w_ref, b_ref, o_ref):
    o_ref[...] = _hyplinear_logmap(x_ref[...], w_ref[...], b_ref[...])


def _mid_body(a_ref, w_ref, b_ref, o_ref):
    h2 = _proj(_expmap0(a_ref[0] + a_ref[1]))
    t = jnp.maximum(_logmap0(h2), 0.0)
    h = _proj(_expmap0(t))
    xt2 = _hyplinear_logmap(h, w_ref[...], b_ref[...])
    d = xt2.shape[-1] // 2
    o_ref[...] = jnp.stack([xt2[:, :d], xt2[:, d:]], axis=0)


def _post_body(a_ref, w_ref, b_ref, o_ref):
    a = jnp.concatenate([a_ref[0], a_ref[1]], axis=-1)
    h2 = _proj(_expmap0(a))
    t = _logmap0(h2)
    h = _proj(_expmap0(t))
    ht = _logmap0(h)
    o_ref[...] = (
        jnp.dot(ht, w_ref[...], preferred_element_type=jnp.float32) + b_ref[...]
    )


def _tc_stage(body, x, wt, b_row, d_out, blk=1000, out_stacked=False):
    if x.ndim == 3:
        n = x.shape[1]
        x_spec = pl.BlockSpec((2, blk, x.shape[2]), lambda i: (0, i, 0))
    else:
        n = x.shape[0]
        x_spec = pl.BlockSpec((blk, x.shape[1]), lambda i: (i, 0))
    if out_stacked:
        out_spec = pl.BlockSpec((2, blk, d_out // 2), lambda i: (0, i, 0))
        out_shape = jax.ShapeDtypeStruct((2, n, d_out // 2), jnp.float32)
    else:
        out_spec = pl.BlockSpec((blk, d_out), lambda i: (i, 0))
        out_shape = jax.ShapeDtypeStruct((n, d_out), jnp.float32)
    return pl.pallas_call(
        body,
        grid=(n // blk,),
        in_specs=[
            x_spec,
            pl.BlockSpec(wt.shape, lambda i: (0, 0)),
            pl.BlockSpec(b_row.shape, lambda i: (0, 0)),
        ],
        out_specs=out_spec,
        out_shape=out_shape,
    )(x, wt, b_row)


_CHUNK = 80
_NSUB = 16


@functools.cache
def _make_spmm(n_nodes, n_chunks, dc, feat_split):
    chunk = _CHUNK
    nsub = _NSUB
    nvec = dc // 16
    assert dc % 128 == 0
    if feat_split:
        per_sub = n_chunks // nsub
        assert n_chunks % nsub == 0
        n_pro = per_sub % 2
        half = (per_sub - n_pro) // 2
        stride = 1
        extra = 0
    else:
        base = n_chunks // (2 * nsub)
        extra = n_chunks - base * 2 * nsub
        assert extra in (0, nsub)
        n_pro = base % 2
        half = (base - n_pro) // 2
        stride = 2 * nsub
    rpt = ((n_nodes // nsub) // 8) * 8
    last_extra = n_nodes - rpt * nsub
    assert 0 <= last_extra < rpt and last_extra % 8 == 0
    mesh = plsc.VectorSubcoreMesh(
        core_axis_name="c", subcore_axis_name="s", num_cores=2, num_subcores=nsub
    )

    @functools.partial(
        pl.kernel,
        out_type=jax.ShapeDtypeStruct((2, n_nodes, dc), jnp.float32),
        mesh=mesh,
        scratch_types=[
            pltpu.VMEM((2, chunk), jnp.int32),
            pltpu.VMEM((2, chunk), jnp.int32),
            pltpu.VMEM((chunk,), jnp.float32),
            pltpu.VMEM((chunk,), jnp.float32),
            pltpu.VMEM((chunk, dc), jnp.float32),
            pltpu.VMEM((chunk, dc), jnp.float32),
            pltpu.VMEM_SHARED((n_nodes, dc), jnp.float32),
            pltpu.SemaphoreType.DMA,
            pltpu.SemaphoreType.DMA,
            pltpu.SemaphoreType.DMA,
            pltpu.SemaphoreType.DMA,
        ],
    )
    def spmm(xt0_hbm, xt1_hbm, ep_hbm, w_hbm, out_hbm,
             ebuf0, ebuf1, wbuf0, wbuf1, rows0, rows1, acc, sg0, sg1, ss0, ss1):
        c = lax.axis_index("c")
        s = lax.axis_index("s")
        start = s * rpt
        gbase = s * per_sub if feat_split else c * nsub + s

        def drain_rows(rows, sem):
            pltpu.make_async_copy(xt0_hbm.at[pl.ds(0, chunk)], rows, sem).wait()

        def start_gather(ebuf, rows, sem):
            if feat_split:
                @pl.when(c == 0)
                def _():
                    pltpu.async_copy(xt0_hbm.at[ebuf.at[0]], rows, sem)

                @pl.when(c == 1)
                def _():
                    pltpu.async_copy(xt1_hbm.at[ebuf.at[0]], rows, sem)
            else:
                pltpu.async_copy(xt0_hbm.at[ebuf.at[0]], rows, sem)

        def scale(wbuf, rows):
            def grp(g, carry):
                wg = wbuf[pl.ds(g * 16, 16)]
                base = g * 16
                for r16 in range(16):
                    wsp = jnp.broadcast_to(wg[r16], (16,))
                    for j in range(nvec):
                        rows[base + r16, pl.ds(j * 16, 16)] = (
                            rows[base + r16, pl.ds(j * 16, 16)] * wsp
                        )
                return carry

            lax.fori_loop(0, chunk // 16, grp, 0)

        def load_chunk(cid, ebuf, wbuf):
            pltpu.sync_copy(ep_hbm.at[cid], ebuf)
            pltpu.sync_copy(w_hbm.at[pl.ds(cid * chunk, chunk)], wbuf)

        def serial_chunk(cid):
            load_chunk(cid, ebuf0, wbuf0)
            start_gather(ebuf0, rows0, sg0)
            drain_rows(rows0, sg0)
            scale(wbuf0, rows0)
            pltpu.sync_copy(rows0, acc.at[ebuf0.at[1]], add=True)

        def zrow(i, carry):
            for j in range(nvec):
                rows0[i, pl.ds(j * 16, 16)] = jnp.zeros((16,), jnp.float32)
            return carry

        lax.fori_loop(0, chunk, zrow, 0)
        nfull = rpt // chunk
        for t in range(nfull):
            pltpu.sync_copy(rows0, acc.at[pl.ds(start + t * chunk, chunk)])
        tail = rpt - nfull * chunk
        if tail:
            @pl.when(s < nsub - 1)
            def _():
                pltpu.sync_copy(rows0.at[pl.ds(0, tail)],
                                acc.at[pl.ds(start + nfull * chunk, tail)])
        ltail = rpt + last_extra - nfull * chunk

        @pl.when(s == nsub - 1)
        def _():
            done = nfull * chunk
            for t in range(ltail // chunk):
                pltpu.sync_copy(rows0,
                                acc.at[pl.ds(start + done + t * chunk, chunk)])
            rem = ltail - (ltail // chunk) * chunk
            if rem:
                pltpu.sync_copy(rows0.at[pl.ds(0, rem)],
                                acc.at[pl.ds(start + done + (ltail // chunk) * chunk,
                                             rem)])
        plsc.subcore_barrier()

        for p in range(n_pro):
            serial_chunk(gbase + p * stride)

        if half > 0:
            i0 = n_pro
            load_chunk(gbase + i0 * stride, ebuf0, wbuf0)
            start_gather(ebuf0, rows0, sg0)
            load_chunk(gbase + (i0 + 1) * stride, ebuf1, wbuf1)
            start_gather(ebuf1, rows1, sg1)

            def body(k, carry):
                i = i0 + 2 * k
                drain_rows(rows0, sg0)
                scale(wbuf0, rows0)
                pltpu.async_copy(rows0, acc.at[ebuf0.at[1]], ss0, add=True)
                drain_rows(rows1, sg1)
                scale(wbuf1, rows1)
                pltpu.async_copy(rows1, acc.at[ebuf1.at[1]], ss1, add=True)

                @pl.when(k < half - 1)
                def _():
                    drain_rows(rows0, ss0)
                    load_chunk(gbase + (i + 2) * stride, ebuf0, wbuf0)
                    start_gather(ebuf0, rows0, sg0)
                    drain_rows(rows1, ss1)
                    load_chunk(gbase + (i + 3) * stride, ebuf1, wbuf1)
                    start_gather(ebuf1, rows1, sg1)

                @pl.when(k == half - 1)
                def _():
                    drain_rows(rows0, ss0)
                    drain_rows(rows1, ss1)
                return carry

            lax.fori_loop(0, half, body, 0)

        if extra:
            @pl.when(c == 0)
            def _():
                serial_chunk(2 * half * stride + n_pro * stride + gbase)
        plsc.subcore_barrier()

        for cc in range(2):
            @pl.when(jnp.logical_and(c == cc, s < nsub - 1))
            def _(cc=cc):
                pltpu.sync_copy(acc.at[pl.ds(start, rpt)],
                                out_hbm.at[cc, pl.ds(start, rpt)])

            @pl.when(jnp.logical_and(c == cc, s == nsub - 1))
            def _(cc=cc):
                pltpu.sync_copy(acc.at[pl.ds(start, rpt + last_extra)],
                                out_hbm.at[cc, pl.ds(start, rpt + last_extra)])

    return spmm


def _pack_edges(src, dst):
    n_chunks = src.shape[0] // _CHUNK
    return jnp.stack(
        [src.reshape(n_chunks, _CHUNK), dst.reshape(n_chunks, _CHUNK)], axis=1
    )


def _spmm_feat_split(xt_halves, ep, w):
    _, n, dc = xt_halves.shape
    f = _make_spmm(n, ep.shape[0], dc, True)
    return f(xt_halves[0], xt_halves[1], ep, w)


def _spmm_edge_split(xt, ep, w):
    n, d = xt.shape
    f = _make_spmm(n, ep.shape[0], d, False)
    return f(xt, xt, ep, w)


def kernel(x, edge_index, edge_weight, W1, b1, W2, b2, W_out, b_out):
    d_hid = W1.shape[0]
    d_feat = W2.shape[0]
    n_out = W_out.shape[0]
    src = edge_index[0].astype(jnp.int32)
    dst = edge_index[1].astype(jnp.int32)
    ep = _pack_edges(src, dst)

    xt1 = _tc_stage(_pre1_body, x, W1.T, b1.reshape(1, -1), d_hid)
    agg1 = _spmm_edge_split(xt1, ep, edge_weight)
    xt2_halves = _tc_stage(_mid_body, agg1, W2.T, b2.reshape(1, -1), d_feat,
                           out_stacked=True)
    agg2 = _spmm_feat_split(xt2_halves, ep, edge_weight)
    return _tc_stage(_post_body, agg2, W_out.T, b_out.reshape(1, -1), n_out)


# speedup vs baseline: 5.5936x; 1.1316x over previous
import functools

import jax
import jax.numpy as jnp
from jax import lax
from jax.experimental import pallas as pl
from jax.experimental.pallas import tpu as pltpu
from jax.experimental.pallas import tpu_sc as plsc

_MIN_NORM = 1e-15
_EPS = 4e-3


def _artanh(x):
    x = jnp.clip(x, -1.0 + 1e-7, 1.0 - 1e-7)
    return 0.5 * jnp.log((1.0 + x) / (1.0 - x))


def _norm(x):
    return jnp.maximum(jnp.sqrt(jnp.sum(x * x, axis=-1, keepdims=True)), _MIN_NORM)


def _proj(x):
    n = _norm(x)
    maxnorm = 1.0 - _EPS
    return jnp.where(n > maxnorm, x / n * maxnorm, x)


def _expmap0(u):
    n = _norm(u)
    return jnp.tanh(n) * u / n


def _logmap0(p):
    n = _norm(p)
    return _artanh(n) * p / n


def _mobius_add(x, y):
    x2 = jnp.sum(x * x, axis=-1, keepdims=True)
    y2 = jnp.sum(y * y, axis=-1, keepdims=True)
    xy = jnp.sum(x * y, axis=-1, keepdims=True)
    num = (1.0 + 2.0 * xy + y2) * x + (1.0 - x2) * y
    den = 1.0 + 2.0 * xy + x2 * y2
    return num / jnp.maximum(den, _MIN_NORM)


def _mobius_matvec(wt, x):
    xn = _norm(x)
    mx = jnp.dot(x, wt, preferred_element_type=jnp.float32)
    mxn = _norm(mx)
    res = jnp.tanh(mxn / xn * _artanh(xn)) * mx / mxn
    cond = jnp.all(mx == 0.0, axis=-1, keepdims=True)
    return jnp.where(cond, 0.0, res)


def _hyplinear_logmap(h, wt, b_row):
    mv = _proj(_mobius_matvec(wt, h))
    hb = _proj(_expmap0(b_row))
    res = _proj(_mobius_add(mv, hb))
    return _logmap0(res)


def _pre1_body(x_ref, w_ref, b_ref, o_ref):
    o_ref[...] = _hyplinear_logmap(x_ref[...], w_ref[...], b_ref[...])


def _mid_body(a_ref, w_ref, b_ref, o_ref):
    h2 = _proj(_expmap0(a_ref[0] + a_ref[1]))
    t = jnp.maximum(_logmap0(h2), 0.0)
    h = _proj(_expmap0(t))
    xt2 = _hyplinear_logmap(h, w_ref[...], b_ref[...])
    d = xt2.shape[-1] // 2
    o_ref[...] = jnp.stack([xt2[:, :d], xt2[:, d:]], axis=0)


def _post_body(a_ref, w_ref, b_ref, o_ref):
    a = jnp.concatenate([a_ref[0], a_ref[1]], axis=-1)
    h2 = _proj(_expmap0(a))
    t = _logmap0(h2)
    h = _proj(_expmap0(t))
    ht = _logmap0(h)
    o_ref[...] = (
        jnp.dot(ht, w_ref[...], preferred_element_type=jnp.float32) + b_ref[...]
    )


def _tc_stage(body, x, wt, b_row, d_out, blk=1000, out_stacked=False):
    if x.ndim == 3:
        n = x.shape[1]
        x_spec = pl.BlockSpec((2, blk, x.shape[2]), lambda i: (0, i, 0))
    else:
        n = x.shape[0]
        x_spec = pl.BlockSpec((blk, x.shape[1]), lambda i: (i, 0))
    if out_stacked:
        out_spec = pl.BlockSpec((2, blk, d_out // 2), lambda i: (0, i, 0))
        out_shape = jax.ShapeDtypeStruct((2, n, d_out // 2), jnp.float32)
    else:
        out_spec = pl.BlockSpec((blk, d_out), lambda i: (i, 0))
        out_shape = jax.ShapeDtypeStruct((n, d_out), jnp.float32)
    return pl.pallas_call(
        body,
        grid=(n // blk,),
        in_specs=[
            x_spec,
            pl.BlockSpec(wt.shape, lambda i: (0, 0)),
            pl.BlockSpec(b_row.shape, lambda i: (0, 0)),
        ],
        out_specs=out_spec,
        out_shape=out_shape,
    )(x, wt, b_row)


_CHUNK = 80
_NSUB = 16


@functools.cache
def _make_spmm(n_nodes, n_chunks, dc, feat_split):
    chunk = _CHUNK
    nsub = _NSUB
    depth = 4
    nvec = dc // 16
    assert dc % 128 == 0
    if feat_split:
        per = n_chunks // nsub
        assert n_chunks % nsub == 0
        stride = 1
        extra = 0
    else:
        per = n_chunks // (2 * nsub)
        extra = n_chunks - per * 2 * nsub
        assert extra in (0, nsub)
        stride = 2 * nsub
    n_pro = per % depth
    quads = (per - n_pro) // depth
    rpt = ((n_nodes // nsub) // 8) * 8
    last_extra = n_nodes - rpt * nsub
    assert 0 <= last_extra < rpt and last_extra % 8 == 0
    mesh = plsc.VectorSubcoreMesh(
        core_axis_name="c", subcore_axis_name="s", num_cores=2, num_subcores=nsub
    )
    scratch = (
        [pltpu.VMEM((2, chunk), jnp.int32)] * depth
        + [pltpu.VMEM((chunk,), jnp.float32)] * depth
        + [pltpu.VMEM((chunk, dc), jnp.float32)] * depth
        + [pltpu.VMEM_SHARED((n_nodes, dc), jnp.float32)]
        + [pltpu.SemaphoreType.DMA] * (2 * depth)
    )

    @functools.partial(
        pl.kernel,
        out_type=jax.ShapeDtypeStruct((2, n_nodes, dc), jnp.float32),
        mesh=mesh,
        scratch_types=scratch,
    )
    def spmm(xt_hbm, ep_hbm, w_hbm, out_hbm, *scr):
        ebuf = scr[:depth]
        wbuf = scr[depth:2 * depth]
        rows = scr[2 * depth:3 * depth]
        acc = scr[3 * depth]
        sg = scr[3 * depth + 1:3 * depth + 1 + depth]
        ss = scr[3 * depth + 1 + depth:]
        c = lax.axis_index("c")
        s = lax.axis_index("s")
        start = s * rpt
        gbase = s * per if feat_split else c * nsub + s

        def dummy_rows_src():
            if feat_split:
                return xt_hbm.at[0].at[pl.ds(0, chunk)]
            return xt_hbm.at[pl.ds(0, chunk)]

        def drain(rows_q, sem):
            pltpu.make_async_copy(dummy_rows_src(), rows_q, sem).wait()

        def start_gather(ebuf_q, rows_q, sem):
            if feat_split:
                @pl.when(c == 0)
                def _():
                    pltpu.async_copy(xt_hbm.at[0].at[ebuf_q.at[0]], rows_q, sem)

                @pl.when(c == 1)
                def _():
                    pltpu.async_copy(xt_hbm.at[1].at[ebuf_q.at[0]], rows_q, sem)
            else:
                pltpu.async_copy(xt_hbm.at[ebuf_q.at[0]], rows_q, sem)

        def scale(wbuf_q, rows_q):
            def grp(g, carry):
                wg = wbuf_q[pl.ds(g * 16, 16)]
                base = g * 16
                for r16 in range(16):
                    wsp = jnp.broadcast_to(wg[r16], (16,))
                    for j in range(nvec):
                        rows_q[base + r16, pl.ds(j * 16, 16)] = (
                            rows_q[base + r16, pl.ds(j * 16, 16)] * wsp
                        )
                return carry

            lax.fori_loop(0, chunk // 16, grp, 0)

        def load_chunk(cid, ebuf_q, wbuf_q):
            pltpu.sync_copy(ep_hbm.at[cid], ebuf_q)
            pltpu.sync_copy(w_hbm.at[pl.ds(cid * chunk, chunk)], wbuf_q)

        def serial_chunk(cid):
            load_chunk(cid, ebuf[0], wbuf[0])
            start_gather(ebuf[0], rows[0], sg[0])
            drain(rows[0], sg[0])
            scale(wbuf[0], rows[0])
            pltpu.sync_copy(rows[0], acc.at[ebuf[0].at[1]], add=True)

        def zrow(i, carry):
            for j in range(nvec):
                rows[0][i, pl.ds(j * 16, 16)] = jnp.zeros((16,), jnp.float32)
            return carry

        lax.fori_loop(0, chunk, zrow, 0)
        nfull = rpt // chunk
        for t in range(nfull):
            pltpu.sync_copy(rows[0], acc.at[pl.ds(start + t * chunk, chunk)])
        tail = rpt - nfull * chunk
        if tail:
            @pl.when(s < nsub - 1)
            def _():
                pltpu.sync_copy(rows[0].at[pl.ds(0, tail)],
                                acc.at[pl.ds(start + nfull * chunk, tail)])
        ltail = rpt + last_extra - nfull * chunk

        @pl.when(s == nsub - 1)
        def _():
            done = nfull * chunk
            for t in range(ltail // chunk):
                pltpu.sync_copy(rows[0],
                                acc.at[pl.ds(start + done + t * chunk, chunk)])
            rem = ltail - (ltail // chunk) * chunk
            if rem:
                pltpu.sync_copy(rows[0].at[pl.ds(0, rem)],
                                acc.at[pl.ds(start + done + (ltail // chunk) * chunk,
                                             rem)])
        plsc.subcore_barrier()

        for p in range(n_pro):
            serial_chunk(gbase + p * stride)

        i0 = n_pro
        if quads > 0:
            for q in range(depth):
                load_chunk(gbase + (i0 + q) * stride, ebuf[q], wbuf[q])
                start_gather(ebuf[q], rows[q], sg[q])

            def body(k, carry):
                i = i0 + depth * k
                for q in range(depth):
                    drain(rows[q], sg[q])
                    scale(wbuf[q], rows[q])
                    pltpu.async_copy(rows[q], acc.at[ebuf[q].at[1]], ss[q],
                                     add=True)
                    if q >= 1:
                        @pl.when(k < quads - 1)
                        def _(q=q):
                            drain(rows[q - 1], ss[q - 1])
                            load_chunk(gbase + (i + depth + q - 1) * stride,
                                       ebuf[q - 1], wbuf[q - 1])
                            start_gather(ebuf[q - 1], rows[q - 1], sg[q - 1])

                @pl.when(k < quads - 1)
                def _():
                    drain(rows[depth - 1], ss[depth - 1])
                    load_chunk(gbase + (i + 2 * depth - 1) * stride,
                               ebuf[depth - 1], wbuf[depth - 1])
                    start_gather(ebuf[depth - 1], rows[depth - 1], sg[depth - 1])

                @pl.when(k == quads - 1)
                def _():
                    for q in range(depth):
                        drain(rows[q], ss[q])
                return carry

            lax.fori_loop(0, quads, body, 0)

        if extra:
            @pl.when(c == 0)
            def _():
                serial_chunk(per * stride + gbase)
        plsc.subcore_barrier()

        for cc in range(2):
            @pl.when(jnp.logical_and(c == cc, s < nsub - 1))
            def _(cc=cc):
                pltpu.sync_copy(acc.at[pl.ds(start, rpt)],
                                out_hbm.at[cc, pl.ds(start, rpt)])

            @pl.when(jnp.logical_and(c == cc, s == nsub - 1))
            def _(cc=cc):
                pltpu.sync_copy(acc.at[pl.ds(start, rpt + last_extra)],
                                out_hbm.at[cc, pl.ds(start, rpt + last_extra)])

    return spmm


def _pack_edges(src, dst):
    n_chunks = src.shape[0] // _CHUNK
    return jnp.stack(
        [src.reshape(n_chunks, _CHUNK), dst.reshape(n_chunks, _CHUNK)], axis=1
    )


def _spmm_feat_split(xt_halves, ep, w):
    _, n, dc = xt_halves.shape
    f = _make_spmm(n, ep.shape[0], dc, True)
    return f(xt_halves, ep, w)


def _spmm_edge_split(xt, ep, w):
    n, d = xt.shape
    f = _make_spmm(n, ep.shape[0], d, False)
    return f(xt, ep, w)


def kernel(x, edge_index, edge_weight, W1, b1, W2, b2, W_out, b_out):
    d_hid = W1.shape[0]
    d_feat = W2.shape[0]
    n_out = W_out.shape[0]
    src = edge_index[0].astype(jnp.int32)
    dst = edge_index[1].astype(jnp.int32)
    ep = _pack_edges(src, dst)

    xt1 = _tc_stage(_pre1_body, x, W1.T, b1.reshape(1, -1), d_hid)
    agg1 = _spmm_edge_split(xt1, ep, edge_weight)
    xt2_halves = _tc_stage(_mid_body, agg1, W2.T, b2.reshape(1, -1), d_feat,
                           out_stacked=True)
    agg2 = _spmm_feat_split(xt2_halves, ep, edge_weight)
    return _tc_stage(_post_body, agg2, W_out.T, b_out.reshape(1, -1), n_out)
